# baseline (speedup 1.0000x reference)
import functools

import jax
import jax.numpy as jnp
from jax import lax
from jax.experimental import pallas as pl
from jax.experimental.pallas import tpu as pltpu
from jax.experimental.pallas import tpu_sc as plsc

_MAXREL = 128
_LQ = 2048
_LK = 2048
_D = 64
_GROWS = 4096
_BAND_LO = _LQ - _MAXREL
_NSHIFT = 8
_NWORKERS = 32
_ROWS_PER_W = _LQ // _NWORKERS
_DEPTH = 4


def _build_band_all(table_ref, g_ref):
    row0 = table_ref[0:1, :]
    row_last = table_ref[2 * _MAXREL : 2 * _MAXREL + 1, :]
    band = table_ref[0 : 2 * _MAXREL, :]
    ch = 256
    for m in range(_NSHIFT):
        head = _BAND_LO - m
        for k in range(0, head, ch):
            n = min(ch, head - k)
            g_ref[m, k : k + n, :] = jnp.broadcast_to(row0, (n, _D))
        g_ref[m, head : head + 2 * _MAXREL, :] = band
        for k in range(head + 2 * _MAXREL, _GROWS, ch):
            n = min(ch, _GROWS - k)
            g_ref[m, k : k + n, :] = jnp.broadcast_to(row_last, (n, _D))


def _expand_table(table):
    return pl.pallas_call(
        _build_band_all,
        out_shape=jax.ShapeDtypeStruct((_NSHIFT, _GROWS, _D), jnp.float32),
    )(table)


@functools.cache
def _sc_stream_fn():
    mesh = plsc.VectorSubcoreMesh(core_axis_name="c", subcore_axis_name="s")
    return pl.kernel(
        _sc_stream_body,
        out_type=jax.ShapeDtypeStruct((_LQ, _LK, _D), jnp.float32),
        mesh=mesh,
        scratch_types=[pltpu.SemaphoreType.DMA],
    )


def _sc_stream_body(g_hbm, out_hbm, sem):
    wid = lax.axis_index("s") * 2 + lax.axis_index("c")
    base = wid * _ROWS_PER_W

    def _copy(i):
        start = _LQ - i
        m = lax.rem(start, _NSHIFT)
        a = pl.multiple_of(start - m, _NSHIFT)
        return pltpu.make_async_copy(
            g_hbm.at[m, pl.ds(a, _LQ), :], out_hbm.at[i], sem
        )

    def _issue(r, carry):
        @pl.when(r >= _DEPTH)
        def _wait_oldest():
            _copy(base + r - _DEPTH).wait()

        _copy(base + r).start()
        return carry

    lax.fori_loop(0, _ROWS_PER_W, _issue, 0)

    def _drain(r, carry):
        _copy(base + r).wait()
        return carry

    lax.fori_loop(_ROWS_PER_W - _DEPTH, _ROWS_PER_W, _drain, 0)


def kernel(length_q, length_k, embeddings_table):
    return _sc_stream_fn()(_expand_table(embeddings_table))


# speedup vs baseline: 26.3293x; 26.3293x over previous
import functools

import jax
import jax.numpy as jnp
from jax import lax
from jax.experimental import pallas as pl
from jax.experimental.pallas import tpu as pltpu
from jax.experimental.pallas import tpu_sc as plsc

_MAXREL = 128
_LQ = 2048
_LK = 2048
_D = 64
_GROWS = 4104
_BAND_LO = _LQ - _MAXREL
_BAND_HI = _BAND_LO + 2 * _MAXREL
_NWORKERS = 32
_ROWS_PER_W = _LQ // _NWORKERS
_DEPTH = 4


def _build_band(table_ref, g_ref):
    row0 = table_ref[0:1, :]
    row_last = table_ref[2 * _MAXREL : 2 * _MAXREL + 1, :]
    ch = 256
    for k in range(0, _BAND_LO, ch):
        n = min(ch, _BAND_LO - k)
        g_ref[k : k + n, :] = jnp.broadcast_to(row0, (n, _D))
    g_ref[_BAND_LO:_BAND_HI, :] = table_ref[0 : 2 * _MAXREL, :]
    for k in range(_BAND_HI, _GROWS, ch):
        n = min(ch, _GROWS - k)
        g_ref[k : k + n, :] = jnp.broadcast_to(row_last, (n, _D))


def _expand_table(table):
    return pl.pallas_call(
        _build_band,
        out_shape=jax.ShapeDtypeStruct((_GROWS, _D), jnp.float32),
    )(table)


@functools.cache
def _sc_stream_fn():
    mesh = plsc.VectorSubcoreMesh(core_axis_name="c", subcore_axis_name="s")
    return pl.kernel(
        _sc_stream_body,
        out_type=jax.ShapeDtypeStruct((_LQ, _LK, _D), jnp.float32),
        mesh=mesh,
        scratch_types=[
            pltpu.VMEM_SHARED((_GROWS, _D), jnp.float32),
            pltpu.SemaphoreType.DMA,
        ],
    )


def _sc_stream_body(g_hbm, out_hbm, g_sh, sem):
    c = lax.axis_index("c")
    s = lax.axis_index("s")
    wid = c * 16 + s
    base = wid * _ROWS_PER_W

    @pl.when(s == 0)
    def _stage():
        pltpu.sync_copy(g_hbm, g_sh)

    plsc.subcore_barrier()

    def _copy(i):
        start = pl.multiple_of(_LQ - i, 1)
        return pltpu.make_async_copy(
            g_sh.at[pl.ds(start, _LQ), :], out_hbm.at[i], sem
        )

    def _issue(r, carry):
        @pl.when(r >= _DEPTH)
        def _wait_oldest():
            _copy(base + r - _DEPTH).wait()

        _copy(base + r).start()
        return carry

    lax.fori_loop(0, _ROWS_PER_W, _issue, 0)

    def _drain(r, carry):
        _copy(base + r).wait()
        return carry

    lax.fori_loop(_ROWS_PER_W - _DEPTH, _ROWS_PER_W, _drain, 0)


def kernel(length_q, length_k, embeddings_table):
    return _sc_stream_fn()(_expand_table(embeddings_table))


# speedup vs baseline: 26.4385x; 1.0042x over previous
import functools

import jax
import jax.numpy as jnp
from jax import lax
from jax.experimental import pallas as pl
from jax.experimental.pallas import tpu as pltpu
from jax.experimental.pallas import tpu_sc as plsc

_MAXREL = 128
_LQ = 2048
_LK = 2048
_D = 64
_GROWS = 4104
_BAND_LO = _LQ - _MAXREL
_BAND_HI = _BAND_LO + 2 * _MAXREL
_NWORKERS = 32
_ROWS_PER_W = _LQ // _NWORKERS
_DEPTH = 8


def _build_band(table_ref, g_ref):
    row0 = table_ref[0:1, :]
    row_last = table_ref[2 * _MAXREL : 2 * _MAXREL + 1, :]
    ch = 256
    for k in range(0, _BAND_LO, ch):
        n = min(ch, _BAND_LO - k)
        g_ref[k : k + n, :] = jnp.broadcast_to(row0, (n, _D))
    g_ref[_BAND_LO:_BAND_HI, :] = table_ref[0 : 2 * _MAXREL, :]
    for k in range(_BAND_HI, _GROWS, ch):
        n = min(ch, _GROWS - k)
        g_ref[k : k + n, :] = jnp.broadcast_to(row_last, (n, _D))


def _expand_table(table):
    return pl.pallas_call(
        _build_band,
        out_shape=jax.ShapeDtypeStruct((_GROWS, _D), jnp.float32),
    )(table)


@functools.cache
def _sc_stream_fn():
    mesh = plsc.VectorSubcoreMesh(core_axis_name="c", subcore_axis_name="s")
    return pl.kernel(
        _sc_stream_body,
        out_type=jax.ShapeDtypeStruct((_LQ, _LK, _D), jnp.float32),
        mesh=mesh,
        scratch_types=[
            pltpu.VMEM_SHARED((_GROWS, _D), jnp.float32),
            pltpu.SemaphoreType.DMA,
        ],
    )


def _sc_stream_body(g_hbm, out_hbm, g_sh, sem):
    c = lax.axis_index("c")
    s = lax.axis_index("s")
    wid = c * 16 + s
    base = wid * _ROWS_PER_W

    @pl.when(s == 0)
    def _stage():
        pltpu.sync_copy(g_hbm, g_sh)

    plsc.subcore_barrier()

    def _copy(i):
        start = pl.multiple_of(_LQ - i, 1)
        return pltpu.make_async_copy(
            g_sh.at[pl.ds(start, _LQ), :], out_hbm.at[i], sem
        )

    def _issue(r, carry):
        @pl.when(r >= _DEPTH)
        def _wait_oldest():
            _copy(base + r - _DEPTH).wait()

        _copy(base + r).start()
        return carry

    lax.fori_loop(0, _ROWS_PER_W, _issue, 0)

    def _drain(r, carry):
        _copy(base + r).wait()
        return carry

    lax.fori_loop(_ROWS_PER_W - _DEPTH, _ROWS_PER_W, _drain, 0)


def kernel(length_q, length_k, embeddings_table):
    return _sc_stream_fn()(_expand_table(embeddings_table))


# speedup vs baseline: 32.2411x; 1.2195x over previous
import jax
import jax.numpy as jnp
from jax.experimental import pallas as pl
from jax.experimental.pallas import tpu as pltpu

_MAXREL = 128
_LQ = 2048
_LK = 2048
_D = 64
_HROWS = 4096
_BAND_LO = _LQ - _MAXREL
_RB = 8
_NBUF = 8


def _fused(table_ref, out_ref, h, sems):
    row0 = table_ref[0:1, :]
    row_last = table_ref[2 * _MAXREL : 2 * _MAXREL + 1, :]
    ch = 256
    for r in range(_RB):
        shift = _RB - r
        head = _BAND_LO - shift
        for k in range(0, head, ch):
            n = min(ch, head - k)
            h[r, k : k + n, :] = jnp.broadcast_to(row0, (n, _D))
        h[r, head : head + 2 * _MAXREL, :] = table_ref[0 : 2 * _MAXREL, :]
        for k in range(head + 2 * _MAXREL, _HROWS, ch):
            n = min(ch, _HROWS - k)
            h[r, k : k + n, :] = jnp.broadcast_to(row_last, (n, _D))

    nblk = _LQ // _RB

    def _copy(k, slot):
        off = pl.multiple_of(_LQ - _RB - _RB * k, _RB)
        return pltpu.make_async_copy(
            h.at[:, pl.ds(off, _LQ), :],
            out_ref.at[pl.ds(_RB * k, _RB)],
            sems.at[slot],
        )

    def _issue(k, carry):
        @pl.when(k >= _NBUF)
        def _wait_oldest():
            _copy(k - _NBUF, jax.lax.rem(k - _NBUF, _NBUF)).wait()

        _copy(k, jax.lax.rem(k, _NBUF)).start()
        return carry

    jax.lax.fori_loop(0, nblk, _issue, 0)

    def _drain(k, carry):
        _copy(k, jax.lax.rem(k, _NBUF)).wait()
        return carry

    jax.lax.fori_loop(nblk - _NBUF, nblk, _drain, 0)


def _impl(table, interpret=False):
    return pl.pallas_call(
        _fused,
        out_shape=jax.ShapeDtypeStruct((_LQ, _LK, _D), jnp.float32),
        in_specs=[pl.BlockSpec(memory_space=pltpu.MemorySpace.VMEM)],
        out_specs=pl.BlockSpec(memory_space=pltpu.MemorySpace.HBM),
        scratch_shapes=[
            pltpu.VMEM((_RB, _HROWS, _D), jnp.float32),
            pltpu.SemaphoreType.DMA((_NBUF,)),
        ],
        interpret=interpret,
    )(table)


def kernel(length_q, length_k, embeddings_table):
    return _impl(embeddings_table)
